# baseline (speedup 1.0000x reference)
import functools

import jax
import jax.numpy as jnp
from jax import lax
from jax.experimental import pallas as pl
from jax.experimental.pallas import tpu as pltpu, tpu_sc as plsc

F32 = jnp.float32

NNODE = 10000
NEDGE = 320000
DIN = 128
NHID = 64
NOUT = 64
NHEAD = 8

NROW = 10240
NC = 2
NS = 16
NW = NC * NS
EB = 256
NBT = 40
NBLKT = NW * NBT
NPAIR = NBT // 2
RB = 512
NT = NROW // NS
NTC = 128


def _zero_rows(buf, ncol):
    zv = jnp.zeros((16,), F32)

    def body(i, c):
        for j in range(ncol // 16):
            buf[i, pl.ds(j * 16, 16)] = zv
        return c

    lax.fori_loop(0, NTC, body, 0)


def _zero_acc_slice(zbuf, acc, ss):
    for kk in range(NT // NTC):
        pltpu.sync_copy(zbuf.at[pl.ds(0, NTC)],
                        acc.at[pl.ds(ss * NT + kk * NTC, NTC)])


def _sc_logits(asrc, adst, edg):
    mesh = plsc.VectorSubcoreMesh(core_axis_name="c", subcore_axis_name="s")

    @functools.partial(
        pl.kernel,
        out_type=(
            jax.ShapeDtypeStruct((NBLKT, EB, 16), F32),
            jax.ShapeDtypeStruct((NC, NROW, 16), F32),
        ),
        mesh=mesh,
        compiler_params=pltpu.CompilerParams(use_tc_tiling_on_sc=False),
        scratch_types=(
            [pltpu.VMEM((2, EB), jnp.int32)] * 4
            + [pltpu.VMEM((EB, 16), F32)] * 8
            + [pltpu.VMEM_SHARED((NROW, 16), F32)]
            + [pltpu.SemaphoreType.DMA] * 4
        ),
    )
    def k(asrc_hbm, adst_hbm, edg_hbm, w_hbm, dp_hbm, *scr):
        idxs = scr[0:4]
        ars = scr[4:8]
        brs = scr[8:12]
        acc = scr[12]
        semg = scr[13:17]
        cc = lax.axis_index("c")
        ss = lax.axis_index("s")
        wid = ss * NC + cc
        b0 = wid * NBT

        _zero_rows(ars[0], 16)
        _zero_acc_slice(ars[0], acc, ss)
        plsc.subcore_barrier()

        def start(t, loc, drain):
            idxb, ar, br = idxs[t], ars[t], brs[t]
            blk = b0 + loc
            pltpu.sync_copy(edg_hbm.at[blk], idxb)
            pltpu.async_copy(asrc_hbm.at[idxb.at[0]], ar, semg[t])
            pltpu.async_copy(adst_hbm.at[idxb.at[1]], br, semg[t])

        def process(t, loc):
            idxb, ar, br = idxs[t], ars[t], brs[t]
            blk = b0 + loc
            pltpu.make_async_copy(asrc_hbm.at[idxb.at[0]], ar, semg[t]).wait()
            pltpu.make_async_copy(adst_hbm.at[idxb.at[1]], br, semg[t]).wait()

            @plsc.parallel_loop(0, EB, unroll=4)
            def _(i):
                e = ar[i, :] + br[i, :]
                ar[i, :] = jnp.exp(jnp.where(e > 0.0, e, 0.2 * e))

            pltpu.sync_copy(ar, acc.at[idxb.at[1]], add=True)
            pltpu.sync_copy(ar, w_hbm.at[blk])

        start(0, 0, False)
        start(1, 1, False)

        def quad(j, carry):
            b = 4 * j
            start(2, b + 2, True)
            process(0, b)
            start(3, b + 3, True)
            process(1, b + 1)

            @pl.when(b + 4 < NBT)
            def _():
                start(0, b + 4, True)

            process(2, b + 2)

            @pl.when(b + 5 < NBT)
            def _():
                start(1, b + 5, True)

            process(3, b + 3)
            return carry

        lax.fori_loop(0, NBT // 4, quad, 0)
        plsc.subcore_barrier()
        pltpu.sync_copy(acc.at[pl.ds(ss * NT, NT)],
                        dp_hbm.at[cc, pl.ds(ss * NT, NT)])

    return k(asrc, adst, edg)


def _sc_agg(hs, w3, edg, feat, lanes_of):
    nch = len(hs)
    mesh = plsc.VectorSubcoreMesh(core_axis_name="c", subcore_axis_name="s")

    @functools.partial(
        pl.kernel,
        out_type=jax.ShapeDtypeStruct((NC, nch, NROW, feat), F32),
        mesh=mesh,
        compiler_params=pltpu.CompilerParams(use_tc_tiling_on_sc=False),
        scratch_types=(
            [pltpu.VMEM((2, EB), jnp.int32)] * 4
            + [pltpu.VMEM((EB, 16), F32)] * 4
            + [pltpu.VMEM((EB, feat), F32)] * 4
            + [pltpu.VMEM_SHARED((NROW, feat), F32)]
            + [pltpu.SemaphoreType.DMA] * 4
        ),
    )
    def k(*refs):
        h_hbms = refs[:nch]
        w_hbm, edg_hbm, outp_hbm = refs[nch:nch + 3]
        scr = refs[nch + 3:]
        idxs = scr[0:4]
        wrs = scr[4:8]
        hrs = scr[8:12]
        acc = scr[12]
        semg = scr[13:17]
        cc = lax.axis_index("c")
        ss = lax.axis_index("s")
        wid = ss * NC + cc
        b0 = wid * NBT

        for c in range(nch):
            lanes = lanes_of(c)

            _zero_rows(hrs[0], feat)
            _zero_acc_slice(hrs[0], acc, ss)
            plsc.subcore_barrier()

            def start(t, loc, drain, c=c):
                idxb, wr, hr = idxs[t], wrs[t], hrs[t]
                blk = b0 + loc
                pltpu.sync_copy(edg_hbm.at[blk], idxb)
                pltpu.async_copy(w_hbm.at[blk], wr, semg[t])
                pltpu.async_copy(h_hbms[c].at[idxb.at[0]], hr, semg[t])

            def process(t, loc, c=c, lanes=lanes):
                idxb, wr, hr = idxs[t], wrs[t], hrs[t]
                pltpu.make_async_copy(w_hbm.at[0], wr, semg[t]).wait()
                pltpu.make_async_copy(h_hbms[c].at[idxb.at[0]], hr,
                                      semg[t]).wait()

                @plsc.parallel_loop(0, EB, unroll=2)
                def _(i):
                    wv = wr[i, :]
                    for j in range(feat // 16):
                        hr[i, pl.ds(j * 16, 16)] = (
                            hr[i, pl.ds(j * 16, 16)] * wv[lanes[j]])

                pltpu.sync_copy(hr, acc.at[idxb.at[1]], add=True)

            start(0, 0, False)
            start(1, 1, False)

            def quad(j, carry):
                b = 4 * j
                start(2, b + 2, True)
                process(0, b)
                start(3, b + 3, True)
                process(1, b + 1)

                @pl.when(b + 4 < NBT)
                def _():
                    start(0, b + 4, True)

                process(2, b + 2)

                @pl.when(b + 5 < NBT)
                def _():
                    start(1, b + 5, True)

                process(3, b + 3)
                return carry

            lax.fori_loop(0, NBT // 4, quad, 0)
            plsc.subcore_barrier()
            pltpu.sync_copy(acc.at[pl.ds(ss * NT, NT)],
                            outp_hbm.at[cc, c, pl.ds(ss * NT, NT)])

    return k(*hs, w3, edg)


def _tc_dense1(x, W1, A1s, A1d):
    grid = (NROW // RB,)
    return pl.pallas_call(
        _dense1_body,
        grid=grid,
        in_specs=[
            pl.BlockSpec((RB, DIN), lambda i: (i, 0)),
            pl.BlockSpec((DIN, NHEAD * NHID), lambda i: (0, 0)),
            pl.BlockSpec((NHEAD * NHID, 16), lambda i: (0, 0)),
            pl.BlockSpec((NHEAD * NHID, 16), lambda i: (0, 0)),
        ],
        out_specs=[
            pl.BlockSpec((8, RB, 64), lambda i: (0, i, 0)),
            pl.BlockSpec((RB, 16), lambda i: (i, 0)),
            pl.BlockSpec((RB, 16), lambda i: (i, 0)),
        ],
        out_shape=[
            jax.ShapeDtypeStruct((8, NROW, 64), F32),
            jax.ShapeDtypeStruct((NROW, 16), F32),
            jax.ShapeDtypeStruct((NROW, 16), F32),
        ],
    )(x, W1, A1s, A1d)


def _dense1_body(x_ref, w_ref, as_ref, ad_ref, h4_ref, oas_ref, oad_ref):
    h = jnp.dot(x_ref[...], w_ref[...], preferred_element_type=F32)
    oas_ref[...] = jnp.dot(h, as_ref[...], preferred_element_type=F32)
    oad_ref[...] = jnp.dot(h, ad_ref[...], preferred_element_type=F32)
    for c in range(8):
        h4_ref[c] = h[:, c * 64:(c + 1) * 64]


def _tc_dense2(op1, dp1, b1r, W2r, A2s, A2d):
    grid = (NROW // RB,)
    return pl.pallas_call(
        _dense2_body,
        grid=grid,
        in_specs=[
            pl.BlockSpec((NC, 8, RB, 64), lambda i: (0, 0, i, 0)),
            pl.BlockSpec((NC, RB, 16), lambda i: (0, i, 0)),
            pl.BlockSpec((8, 64), lambda i: (0, 0)),
            pl.BlockSpec((8, 64, NOUT), lambda i: (0, 0, 0)),
            pl.BlockSpec((NOUT, 16), lambda i: (0, 0)),
            pl.BlockSpec((NOUT, 16), lambda i: (0, 0)),
        ],
        out_specs=[
            pl.BlockSpec((RB, NOUT), lambda i: (i, 0)),
            pl.BlockSpec((RB, 16), lambda i: (i, 0)),
            pl.BlockSpec((RB, 16), lambda i: (i, 0)),
        ],
        out_shape=[
            jax.ShapeDtypeStruct((NROW, NOUT), F32),
            jax.ShapeDtypeStruct((NROW, 16), F32),
            jax.ShapeDtypeStruct((NROW, 16), F32),
        ],
    )(op1, dp1, b1r, W2r, A2s, A2d)


def _dense2_body(op_ref, dp_ref, b1_ref, w2_ref, as_ref, ad_ref,
                 h2_ref, oas_ref, oad_ref):
    r = 1.0 / (dp_ref[0] + dp_ref[1] + 1e-16)
    h2 = None
    for c in range(8):
        pc = op_ref[0, c] + op_ref[1, c]
        hr = jnp.maximum(pc * r[:, c:c + 1] + b1_ref[c][None, :], 0.0)
        d = jnp.dot(hr, w2_ref[c], preferred_element_type=F32)
        h2 = d if h2 is None else h2 + d
    h2_ref[...] = h2
    oas_ref[...] = jnp.dot(h2, as_ref[...], preferred_element_type=F32)
    oad_ref[...] = jnp.dot(h2, ad_ref[...], preferred_element_type=F32)


def _tc_final(op2, dp2, b2r):
    grid = (NROW // RB,)
    return pl.pallas_call(
        _final_body,
        grid=grid,
        in_specs=[
            pl.BlockSpec((NC, RB, NOUT), lambda i: (0, i, 0)),
            pl.BlockSpec((NC, RB, 16), lambda i: (0, i, 0)),
            pl.BlockSpec((1, NOUT), lambda i: (0, 0)),
        ],
        out_specs=pl.BlockSpec((RB, NOUT), lambda i: (i, 0)),
        out_shape=jax.ShapeDtypeStruct((NROW, NOUT), F32),
    )(op2, dp2, b2r)


def _final_body(op_ref, dp_ref, b2_ref, o_ref):
    r = 1.0 / (dp_ref[0, :, 0:1] + dp_ref[1, :, 0:1] + 1e-16)
    y = (op_ref[0] + op_ref[1]) * r + b2_ref[...]
    m = jnp.max(y, axis=1, keepdims=True)
    z = y - m
    o_ref[...] = z - jnp.log(jnp.sum(jnp.exp(z), axis=1, keepdims=True))


def kernel(x, edge_index, W1, att_src1, att_dst1, b1, W2, att_src2,
           att_dst2, b2):
    npad = NBLKT * EB - NEDGE
    pad = jnp.full((npad,), NROW - 1, jnp.int32)
    srcp = jnp.concatenate([edge_index[0].astype(jnp.int32), pad])
    dstp = jnp.concatenate([edge_index[1].astype(jnp.int32), pad])
    edg = jnp.stack([srcp.reshape(NBLKT, EB), dstp.reshape(NBLKT, EB)],
                    axis=1)

    eye816 = jnp.eye(NHEAD, 16, dtype=F32)
    A1s = (att_src1[:, :, None] * eye816[:, None, :]).reshape(NHEAD * NHID, 16)
    A1d = (att_dst1[:, :, None] * eye816[:, None, :]).reshape(NHEAD * NHID, 16)
    A2s = att_src2.T * jnp.eye(1, 16, dtype=F32)
    A2d = att_dst2.T * jnp.eye(1, 16, dtype=F32)
    b1r = b1.reshape(8, 64)
    W2r = W2.reshape(8, 64, NOUT)
    b2r = b2.reshape(1, NOUT)
    xp = jnp.pad(x, ((0, NROW - NNODE), (0, 0)))

    h4, as1, ad1 = _tc_dense1(xp, W1, A1s, A1d)
    w1, dp1 = _sc_logits(as1, ad1, edg)
    hs1 = tuple(h4[c] for c in range(8))
    lanes1 = lambda c: [c] * 4
    op1 = _sc_agg(hs1, w1, edg, 64, lanes1)

    h2, as2, ad2 = _tc_dense2(op1, dp1, b1r, W2r, A2s, A2d)
    w2, dp2 = _sc_logits(as2, ad2, edg)
    lanes2 = lambda c: [0] * 4
    op2 = _sc_agg((h2,), w2, edg, NOUT, lanes2)

    return _tc_final(op2.reshape(NC, NROW, NOUT), dp2, b2r)[:NNODE]


# speedup vs baseline: 1.4127x; 1.4127x over previous
import functools

import jax
import jax.numpy as jnp
from jax import lax
from jax.experimental import pallas as pl
from jax.experimental.pallas import tpu as pltpu, tpu_sc as plsc

F32 = jnp.float32

NNODE = 10000
NEDGE = 320000
DIN = 128
NHID = 64
NOUT = 64
NHEAD = 8

NROW = 10240
NC = 2
NS = 16
NW = NC * NS
EB = 128
NBT = 80
NBLKT = NW * NBT
NPAIR = NBT // 2
RB = 512
NT = NROW // NS
NTC = 128


def _zero_rows(buf, ncol):
    zv = jnp.zeros((16,), F32)

    def body(i, c):
        for j in range(ncol // 16):
            buf[i, pl.ds(j * 16, 16)] = zv
        return c

    lax.fori_loop(0, NTC, body, 0)


def _zero_acc_slice(zbuf, acc, ss):
    for kk in range(NT // NTC):
        pltpu.sync_copy(zbuf.at[pl.ds(0, NTC)],
                        acc.at[pl.ds(ss * NT + kk * NTC, NTC)])


def _sc_logits(asrc, adst, edg):
    mesh = plsc.VectorSubcoreMesh(core_axis_name="c", subcore_axis_name="s")

    @functools.partial(
        pl.kernel,
        out_type=(
            jax.ShapeDtypeStruct((NBLKT, EB, 16), F32),
            jax.ShapeDtypeStruct((NC, NROW, 16), F32),
        ),
        mesh=mesh,
        compiler_params=pltpu.CompilerParams(use_tc_tiling_on_sc=False),
        scratch_types=(
            [pltpu.VMEM((2, EB), jnp.int32)] * 4
            + [pltpu.VMEM((EB, 16), F32)] * 8
            + [pltpu.VMEM_SHARED((NROW, 16), F32)]
            + [pltpu.SemaphoreType.DMA] * 4
        ),
    )
    def k(asrc_hbm, adst_hbm, edg_hbm, w_hbm, dp_hbm, *scr):
        idxs = scr[0:4]
        ars = scr[4:8]
        brs = scr[8:12]
        acc = scr[12]
        semg = scr[13:17]
        cc = lax.axis_index("c")
        ss = lax.axis_index("s")
        wid = ss * NC + cc
        b0 = wid * NBT

        _zero_rows(ars[0], 16)
        _zero_acc_slice(ars[0], acc, ss)
        plsc.subcore_barrier()

        def start(t, loc, drain):
            idxb, ar, br = idxs[t], ars[t], brs[t]
            blk = b0 + loc
            pltpu.sync_copy(edg_hbm.at[blk], idxb)
            pltpu.async_copy(asrc_hbm.at[idxb.at[0]], ar, semg[t])
            pltpu.async_copy(adst_hbm.at[idxb.at[1]], br, semg[t])

        def process(t, loc):
            idxb, ar, br = idxs[t], ars[t], brs[t]
            blk = b0 + loc
            pltpu.make_async_copy(asrc_hbm.at[idxb.at[0]], ar, semg[t]).wait()
            pltpu.make_async_copy(adst_hbm.at[idxb.at[1]], br, semg[t]).wait()

            @plsc.parallel_loop(0, EB, unroll=4)
            def _(i):
                e = ar[i, :] + br[i, :]
                ar[i, :] = jnp.exp(jnp.where(e > 0.0, e, 0.2 * e))

            pltpu.sync_copy(ar, acc.at[idxb.at[1]], add=True)
            pltpu.sync_copy(ar, w_hbm.at[blk])

        start(0, 0, False)
        start(1, 1, False)

        def quad(j, carry):
            b = 4 * j
            start(2, b + 2, True)
            process(0, b)
            start(3, b + 3, True)
            process(1, b + 1)

            @pl.when(b + 4 < NBT)
            def _():
                start(0, b + 4, True)

            process(2, b + 2)

            @pl.when(b + 5 < NBT)
            def _():
                start(1, b + 5, True)

            process(3, b + 3)
            return carry

        lax.fori_loop(0, NBT // 4, quad, 0)
        plsc.subcore_barrier()
        pltpu.sync_copy(acc.at[pl.ds(ss * NT, NT)],
                        dp_hbm.at[cc, pl.ds(ss * NT, NT)])

    return k(asrc, adst, edg)


def _sc_agg(hs, w3, edg, feat, lanes_of, bf16=False):
    nch = len(hs)
    tdt = jnp.int32 if bf16 else F32
    tcol = feat // 2 if bf16 else feat
    mesh = plsc.VectorSubcoreMesh(core_axis_name="c", subcore_axis_name="s")

    @functools.partial(
        pl.kernel,
        out_type=jax.ShapeDtypeStruct((NC, nch, NROW, feat), F32),
        mesh=mesh,
        compiler_params=pltpu.CompilerParams(use_tc_tiling_on_sc=False),
        scratch_types=(
            [pltpu.VMEM((2, EB), jnp.int32)] * 4
            + [pltpu.VMEM((EB, 16), F32)] * 4
            + [pltpu.VMEM((EB, tcol), tdt)] * 4
            + [pltpu.VMEM((EB, feat), F32)]
            + [pltpu.VMEM_SHARED((NROW, feat), F32)]
            + [pltpu.SemaphoreType.DMA] * 4
        ),
    )
    def k(*refs):
        h_hbms = refs[:nch]
        w_hbm, edg_hbm, outp_hbm = refs[nch:nch + 3]
        scr = refs[nch + 3:]
        idxs = scr[0:4]
        wrs = scr[4:8]
        hrs = scr[8:12]
        hof = scr[12]
        acc = scr[13]
        semg = scr[14:18]
        cc = lax.axis_index("c")
        ss = lax.axis_index("s")
        wid = ss * NC + cc
        b0 = wid * NBT

        for c in range(nch):
            lanes = lanes_of(c)

            _zero_rows(hof, feat)
            _zero_acc_slice(hof, acc, ss)
            plsc.subcore_barrier()

            def start(t, loc, drain, c=c):
                idxb, wr, hr = idxs[t], wrs[t], hrs[t]
                blk = b0 + loc
                pltpu.sync_copy(edg_hbm.at[blk], idxb)
                pltpu.async_copy(w_hbm.at[blk], wr, semg[t])
                pltpu.async_copy(h_hbms[c].at[idxb.at[0]], hr, semg[t])

            def process(t, loc, c=c, lanes=lanes):
                idxb, wr, hr = idxs[t], wrs[t], hrs[t]
                pltpu.make_async_copy(w_hbm.at[0], wr, semg[t]).wait()
                pltpu.make_async_copy(h_hbms[c].at[idxb.at[0]], hr,
                                      semg[t]).wait()

                if not bf16:
                    @plsc.parallel_loop(0, EB, unroll=2)
                    def _(i):
                        wv = wr[i, :]
                        for j in range(feat // 16):
                            hr[i, pl.ds(j * 16, 16)] = (
                                hr[i, pl.ds(j * 16, 16)] * wv[lanes[j]])

                    pltpu.sync_copy(hr, acc.at[idxb.at[1]], add=True)
                else:
                    @plsc.parallel_loop(0, EB, unroll=2)
                    def _(i):
                        wv = wr[i, :]
                        for g in range(feat // 32):
                            sc = wv[lanes[2 * g]]
                            vi = hr[i, pl.ds(g * 16, 16)]
                            lo = lax.bitcast_convert_type(
                                jnp.left_shift(vi, 16), F32)
                            hi = lax.bitcast_convert_type(
                                jnp.bitwise_and(vi, jnp.int32(-65536)), F32)
                            hof[i, pl.ds(g * 32, 16)] = lo * sc
                            hof[i, pl.ds(g * 32 + 16, 16)] = hi * sc

                    pltpu.sync_copy(hof, acc.at[idxb.at[1]], add=True)

            start(0, 0, False)
            start(1, 1, False)

            def quad(j, carry):
                b = 4 * j
                start(2, b + 2, True)
                process(0, b)
                start(3, b + 3, True)
                process(1, b + 1)

                @pl.when(b + 4 < NBT)
                def _():
                    start(0, b + 4, True)

                process(2, b + 2)

                @pl.when(b + 5 < NBT)
                def _():
                    start(1, b + 5, True)

                process(3, b + 3)
                return carry

            lax.fori_loop(0, NBT // 4, quad, 0)
            plsc.subcore_barrier()
            pltpu.sync_copy(acc.at[pl.ds(ss * NT, NT)],
                            outp_hbm.at[cc, c, pl.ds(ss * NT, NT)])

    return k(*hs, w3, edg)


def _tc_dense1(x, W1, A1s, A1d):
    grid = (NROW // RB,)
    return pl.pallas_call(
        _dense1_body,
        grid=grid,
        in_specs=[
            pl.BlockSpec((RB, DIN), lambda i: (i, 0)),
            pl.BlockSpec((DIN, NHEAD * NHID), lambda i: (0, 0)),
            pl.BlockSpec((NHEAD * NHID, 16), lambda i: (0, 0)),
            pl.BlockSpec((NHEAD * NHID, 16), lambda i: (0, 0)),
        ],
        out_specs=[
            pl.BlockSpec((8, RB, 64), lambda i: (0, i, 0)),
            pl.BlockSpec((RB, 16), lambda i: (i, 0)),
            pl.BlockSpec((RB, 16), lambda i: (i, 0)),
        ],
        out_shape=[
            jax.ShapeDtypeStruct((8, NROW, 64), jnp.bfloat16),
            jax.ShapeDtypeStruct((NROW, 16), F32),
            jax.ShapeDtypeStruct((NROW, 16), F32),
        ],
    )(x, W1, A1s, A1d)


def _dense1_body(x_ref, w_ref, as_ref, ad_ref, h4_ref, oas_ref, oad_ref):
    h = jnp.dot(x_ref[...], w_ref[...], preferred_element_type=F32)
    oas_ref[...] = jnp.dot(h, as_ref[...], preferred_element_type=F32)
    oad_ref[...] = jnp.dot(h, ad_ref[...], preferred_element_type=F32)
    for c in range(8):
        h4_ref[c] = h[:, c * 64:(c + 1) * 64].astype(jnp.bfloat16)


def _tc_dense2(op1, dp1, b1r, W2r, A2s, A2d):
    grid = (NROW // RB,)
    return pl.pallas_call(
        _dense2_body,
        grid=grid,
        in_specs=[
            pl.BlockSpec((NC, 8, RB, 64), lambda i: (0, 0, i, 0)),
            pl.BlockSpec((NC, RB, 16), lambda i: (0, i, 0)),
            pl.BlockSpec((8, 64), lambda i: (0, 0)),
            pl.BlockSpec((8, 64, NOUT), lambda i: (0, 0, 0)),
            pl.BlockSpec((NOUT, 16), lambda i: (0, 0)),
            pl.BlockSpec((NOUT, 16), lambda i: (0, 0)),
        ],
        out_specs=[
            pl.BlockSpec((RB, NOUT), lambda i: (i, 0)),
            pl.BlockSpec((RB, 16), lambda i: (i, 0)),
            pl.BlockSpec((RB, 16), lambda i: (i, 0)),
        ],
        out_shape=[
            jax.ShapeDtypeStruct((NROW, NOUT), F32),
            jax.ShapeDtypeStruct((NROW, 16), F32),
            jax.ShapeDtypeStruct((NROW, 16), F32),
        ],
    )(op1, dp1, b1r, W2r, A2s, A2d)


def _dense2_body(op_ref, dp_ref, b1_ref, w2_ref, as_ref, ad_ref,
                 h2_ref, oas_ref, oad_ref):
    r = 1.0 / (dp_ref[0] + dp_ref[1] + 1e-16)
    h2 = None
    for c in range(8):
        pc = op_ref[0, c] + op_ref[1, c]
        hr = jnp.maximum(pc * r[:, c:c + 1] + b1_ref[c][None, :], 0.0)
        d = jnp.dot(hr, w2_ref[c], preferred_element_type=F32)
        h2 = d if h2 is None else h2 + d
    h2_ref[...] = h2
    oas_ref[...] = jnp.dot(h2, as_ref[...], preferred_element_type=F32)
    oad_ref[...] = jnp.dot(h2, ad_ref[...], preferred_element_type=F32)


def _tc_final(op2, dp2, b2r):
    grid = (NROW // RB,)
    return pl.pallas_call(
        _final_body,
        grid=grid,
        in_specs=[
            pl.BlockSpec((NC, RB, NOUT), lambda i: (0, i, 0)),
            pl.BlockSpec((NC, RB, 16), lambda i: (0, i, 0)),
            pl.BlockSpec((1, NOUT), lambda i: (0, 0)),
        ],
        out_specs=pl.BlockSpec((RB, NOUT), lambda i: (i, 0)),
        out_shape=jax.ShapeDtypeStruct((NROW, NOUT), F32),
    )(op2, dp2, b2r)


def _final_body(op_ref, dp_ref, b2_ref, o_ref):
    r = 1.0 / (dp_ref[0, :, 0:1] + dp_ref[1, :, 0:1] + 1e-16)
    y = (op_ref[0] + op_ref[1]) * r + b2_ref[...]
    m = jnp.max(y, axis=1, keepdims=True)
    z = y - m
    o_ref[...] = z - jnp.log(jnp.sum(jnp.exp(z), axis=1, keepdims=True))


def kernel(x, edge_index, W1, att_src1, att_dst1, b1, W2, att_src2,
           att_dst2, b2):
    npad = NBLKT * EB - NEDGE
    pad = jnp.full((npad,), NROW - 1, jnp.int32)
    srcp = jnp.concatenate([edge_index[0].astype(jnp.int32), pad])
    dstp = jnp.concatenate([edge_index[1].astype(jnp.int32), pad])
    edg = jnp.stack([srcp.reshape(NBLKT, EB), dstp.reshape(NBLKT, EB)],
                    axis=1)

    eye816 = jnp.eye(NHEAD, 16, dtype=F32)
    A1s = (att_src1[:, :, None] * eye816[:, None, :]).reshape(NHEAD * NHID, 16)
    A1d = (att_dst1[:, :, None] * eye816[:, None, :]).reshape(NHEAD * NHID, 16)
    A2s = att_src2.T * jnp.eye(1, 16, dtype=F32)
    A2d = att_dst2.T * jnp.eye(1, 16, dtype=F32)
    permc = jnp.array([2 * k for k in range(16)]
                      + [2 * k + 1 for k in range(16)]
                      + [32 + 2 * k for k in range(16)]
                      + [33 + 2 * k for k in range(16)], dtype=jnp.int32)
    b1r = b1.reshape(8, 64)[:, permc]
    W2r = W2.reshape(8, 64, NOUT)[:, permc, :]
    b2r = b2.reshape(1, NOUT)
    xp = jnp.pad(x, ((0, NROW - NNODE), (0, 0)))

    h4, as1, ad1 = _tc_dense1(xp, W1, A1s, A1d)
    w1, dp1 = _sc_logits(as1, ad1, edg)
    hs1 = tuple(
        jax.lax.bitcast_convert_type(h4[c].reshape(NROW, 32, 2), jnp.int32)
        for c in range(8))
    lanes1 = lambda c: [c] * 4
    op1 = _sc_agg(hs1, w1, edg, 64, lanes1, bf16=True)

    h2, as2, ad2 = _tc_dense2(op1, dp1, b1r, W2r, A2s, A2d)
    w2, dp2 = _sc_logits(as2, ad2, edg)
    lanes2 = lambda c: [0] * 4
    op2 = _sc_agg((h2,), w2, edg, NOUT, lanes2)

    return _tc_final(op2.reshape(NC, NROW, NOUT), dp2, b2r)[:NNODE]


# speedup vs baseline: 1.5742x; 1.1143x over previous
import functools

import jax
import jax.numpy as jnp
from jax import lax
from jax.experimental import pallas as pl
from jax.experimental.pallas import tpu as pltpu, tpu_sc as plsc

F32 = jnp.float32

NNODE = 10000
NEDGE = 320000
DIN = 128
NHID = 64
NOUT = 64
NHEAD = 8

NROW = 10240
NC = 2
NS = 16
NW = NC * NS
EB = 128
NBT = 80
NBLKT = NW * NBT
NPAIR = NBT // 2
RB = 512
NT = NROW // NS
NTC = 128


def _zero_rows(buf, ncol):
    zv = jnp.zeros((16,), F32)

    def body(i, c):
        for j in range(ncol // 16):
            buf[i, pl.ds(j * 16, 16)] = zv
        return c

    lax.fori_loop(0, NTC, body, 0)


def _zero_acc_slice(zbuf, acc, ss):
    for kk in range(NT // NTC):
        pltpu.sync_copy(zbuf.at[pl.ds(0, NTC)],
                        acc.at[pl.ds(ss * NT + kk * NTC, NTC)])


def _sc_logits(asrc, adst, edg):
    mesh = plsc.VectorSubcoreMesh(core_axis_name="c", subcore_axis_name="s")

    @functools.partial(
        pl.kernel,
        out_type=(
            jax.ShapeDtypeStruct((NBLKT, EB, 16), F32),
            jax.ShapeDtypeStruct((NC, NROW, 16), F32),
        ),
        mesh=mesh,
        compiler_params=pltpu.CompilerParams(use_tc_tiling_on_sc=False),
        scratch_types=(
            [pltpu.VMEM((2, EB), jnp.int32)] * 4
            + [pltpu.VMEM((EB, 16), F32)] * 8
            + [pltpu.VMEM_SHARED((NROW, 16), F32)]
            + [pltpu.SemaphoreType.DMA] * 4
        ),
    )
    def k(asrc_hbm, adst_hbm, edg_hbm, w_hbm, dp_hbm, *scr):
        idxs = scr[0:4]
        ars = scr[4:8]
        brs = scr[8:12]
        acc = scr[12]
        semg = scr[13:17]
        cc = lax.axis_index("c")
        ss = lax.axis_index("s")
        wid = ss * NC + cc
        b0 = wid * NBT

        _zero_rows(ars[0], 16)
        _zero_acc_slice(ars[0], acc, ss)
        plsc.subcore_barrier()

        def start(t, loc, drain):
            idxb, ar, br = idxs[t], ars[t], brs[t]
            blk = b0 + loc
            pltpu.sync_copy(edg_hbm.at[blk], idxb)
            pltpu.async_copy(asrc_hbm.at[idxb.at[0]], ar, semg[t])
            pltpu.async_copy(adst_hbm.at[idxb.at[1]], br, semg[t])

        def process(t, loc):
            idxb, ar, br = idxs[t], ars[t], brs[t]
            blk = b0 + loc
            pltpu.make_async_copy(asrc_hbm.at[idxb.at[0]], ar, semg[t]).wait()
            pltpu.make_async_copy(adst_hbm.at[idxb.at[1]], br, semg[t]).wait()

            @plsc.parallel_loop(0, EB, unroll=4)
            def _(i):
                e = ar[i, :] + br[i, :]
                ar[i, :] = jnp.exp(jnp.where(e > 0.0, e, 0.2 * e))

            pltpu.sync_copy(ar, acc.at[idxb.at[1]], add=True)
            pltpu.sync_copy(ar, w_hbm.at[blk])

        start(0, 0, False)
        start(1, 1, False)

        def quad(j, carry):
            b = 4 * j
            start(2, b + 2, True)
            process(0, b)
            start(3, b + 3, True)
            process(1, b + 1)

            @pl.when(b + 4 < NBT)
            def _():
                start(0, b + 4, True)

            process(2, b + 2)

            @pl.when(b + 5 < NBT)
            def _():
                start(1, b + 5, True)

            process(3, b + 3)
            return carry

        lax.fori_loop(0, NBT // 4, quad, 0)
        plsc.subcore_barrier()
        pltpu.sync_copy(acc.at[pl.ds(ss * NT, NT)],
                        dp_hbm.at[cc, pl.ds(ss * NT, NT)])

    return k(asrc, adst, edg)


def _sc_agg(hs, w3, edg, feat, lanes_of, bf16=False):
    nch = len(hs)
    tdt = jnp.int32 if bf16 else F32
    tcol = feat // 2 if bf16 else feat
    mesh = plsc.VectorSubcoreMesh(core_axis_name="c", subcore_axis_name="s")

    @functools.partial(
        pl.kernel,
        out_type=jax.ShapeDtypeStruct((NC, nch, NROW, feat), F32),
        mesh=mesh,
        compiler_params=pltpu.CompilerParams(use_tc_tiling_on_sc=False),
        scratch_types=(
            [pltpu.VMEM((2, EB), jnp.int32)] * 4
            + [pltpu.VMEM((EB, 16), F32)] * 4
            + [pltpu.VMEM((EB, tcol), tdt)] * 4
            + [pltpu.VMEM((EB, feat), F32)]
            + [pltpu.VMEM_SHARED((NROW, feat), F32)]
            + [pltpu.SemaphoreType.DMA] * 4
        ),
    )
    def k(*refs):
        h_hbms = refs[:nch]
        w_hbm, edg_hbm, outp_hbm = refs[nch:nch + 3]
        scr = refs[nch + 3:]
        idxs = scr[0:4]
        wrs = scr[4:8]
        hrs = scr[8:12]
        hof = scr[12]
        acc = scr[13]
        semg = scr[14:18]
        cc = lax.axis_index("c")
        ss = lax.axis_index("s")
        wid = ss * NC + cc
        b0 = wid * NBT

        for c in range(nch):
            lanes = lanes_of(c)

            _zero_rows(hof, feat)
            _zero_acc_slice(hof, acc, ss)
            plsc.subcore_barrier()

            def start(t, loc, drain, c=c):
                idxb, wr, hr = idxs[t], wrs[t], hrs[t]
                blk = b0 + loc
                pltpu.sync_copy(edg_hbm.at[blk], idxb)
                pltpu.async_copy(w_hbm.at[blk], wr, semg[t])
                pltpu.async_copy(h_hbms[c].at[idxb.at[0]], hr, semg[t])

            def process(t, loc, c=c, lanes=lanes):
                idxb, wr, hr = idxs[t], wrs[t], hrs[t]
                pltpu.make_async_copy(w_hbm.at[0], wr, semg[t]).wait()
                pltpu.make_async_copy(h_hbms[c].at[idxb.at[0]], hr,
                                      semg[t]).wait()

                if not bf16:
                    @plsc.parallel_loop(0, EB, unroll=2)
                    def _(i):
                        wv = wr[i, :]
                        for j in range(feat // 16):
                            hr[i, pl.ds(j * 16, 16)] = (
                                hr[i, pl.ds(j * 16, 16)] * wv[lanes[j]])

                    pltpu.sync_copy(hr, acc.at[idxb.at[1]], add=True)
                else:
                    @plsc.parallel_loop(0, EB, unroll=2)
                    def _(i):
                        wv = wr[i, :]
                        for g in range(feat // 32):
                            sc = wv[lanes[2 * g]]
                            vi = hr[i, pl.ds(g * 16, 16)]
                            lo = lax.bitcast_convert_type(
                                jnp.left_shift(vi, 16), F32)
                            hi = lax.bitcast_convert_type(
                                jnp.bitwise_and(vi, jnp.int32(-65536)), F32)
                            hof[i, pl.ds(g * 32, 16)] = lo * sc
                            hof[i, pl.ds(g * 32 + 16, 16)] = hi * sc

                    pltpu.sync_copy(hof, acc.at[idxb.at[1]], add=True)

            start(0, 0, False)
            start(1, 1, False)

            def quad(j, carry):
                b = 4 * j
                start(2, b + 2, True)
                process(0, b)
                start(3, b + 3, True)
                process(1, b + 1)

                @pl.when(b + 4 < NBT)
                def _():
                    start(0, b + 4, True)

                process(2, b + 2)

                @pl.when(b + 5 < NBT)
                def _():
                    start(1, b + 5, True)

                process(3, b + 3)
                return carry

            lax.fori_loop(0, NBT // 4, quad, 0)
            plsc.subcore_barrier()
            pltpu.sync_copy(acc.at[pl.ds(ss * NT, NT)],
                            outp_hbm.at[cc, c, pl.ds(ss * NT, NT)])

    return k(*hs, w3, edg)


def _tc_dense1(x, W1, A1s, A1d):
    grid = (NROW // RB,)
    return pl.pallas_call(
        _dense1_body,
        grid=grid,
        in_specs=[
            pl.BlockSpec((RB, DIN), lambda i: (i, 0)),
            pl.BlockSpec((DIN, NHEAD * NHID), lambda i: (0, 0)),
            pl.BlockSpec((NHEAD * NHID, 16), lambda i: (0, 0)),
            pl.BlockSpec((NHEAD * NHID, 16), lambda i: (0, 0)),
        ],
        out_specs=[
            pl.BlockSpec((8, RB, 64), lambda i: (0, i, 0)),
            pl.BlockSpec((RB, 16), lambda i: (i, 0)),
            pl.BlockSpec((RB, 16), lambda i: (i, 0)),
        ],
        out_shape=[
            jax.ShapeDtypeStruct((8, NROW, 64), jnp.bfloat16),
            jax.ShapeDtypeStruct((NROW, 16), F32),
            jax.ShapeDtypeStruct((NROW, 16), F32),
        ],
    )(x, W1, A1s, A1d)


def _dense1_body(x_ref, w_ref, as_ref, ad_ref, h4_ref, oas_ref, oad_ref):
    h = jnp.dot(x_ref[...], w_ref[...], preferred_element_type=F32)
    oas_ref[...] = jnp.dot(h, as_ref[...], preferred_element_type=F32)
    oad_ref[...] = jnp.dot(h, ad_ref[...], preferred_element_type=F32)
    for c in range(8):
        h4_ref[c] = h[:, c * 64:(c + 1) * 64].astype(jnp.bfloat16)


def _tc_dense2(op1, dp1, b1r, W2r, A2s, A2d):
    grid = (NROW // RB,)
    return pl.pallas_call(
        _dense2_body,
        grid=grid,
        in_specs=[
            pl.BlockSpec((NC, 8, RB, 64), lambda i: (0, 0, i, 0)),
            pl.BlockSpec((NC, RB, 16), lambda i: (0, i, 0)),
            pl.BlockSpec((8, 64), lambda i: (0, 0)),
            pl.BlockSpec((8, 64, NOUT), lambda i: (0, 0, 0)),
            pl.BlockSpec((NOUT, 16), lambda i: (0, 0)),
            pl.BlockSpec((NOUT, 16), lambda i: (0, 0)),
        ],
        out_specs=[
            pl.BlockSpec((RB, NOUT), lambda i: (i, 0)),
            pl.BlockSpec((RB, 16), lambda i: (i, 0)),
            pl.BlockSpec((RB, 16), lambda i: (i, 0)),
        ],
        out_shape=[
            jax.ShapeDtypeStruct((NROW, NOUT), jnp.bfloat16),
            jax.ShapeDtypeStruct((NROW, 16), F32),
            jax.ShapeDtypeStruct((NROW, 16), F32),
        ],
    )(op1, dp1, b1r, W2r, A2s, A2d)


def _dense2_body(op_ref, dp_ref, b1_ref, w2_ref, as_ref, ad_ref,
                 h2_ref, oas_ref, oad_ref):
    r = 1.0 / (dp_ref[0] + dp_ref[1] + 1e-16)
    h2 = None
    for c in range(8):
        pc = op_ref[0, c] + op_ref[1, c]
        hr = jnp.maximum(pc * r[:, c:c + 1] + b1_ref[c][None, :], 0.0)
        d = jnp.dot(hr, w2_ref[c], preferred_element_type=F32)
        h2 = d if h2 is None else h2 + d
    h2_ref[...] = h2.astype(jnp.bfloat16)
    oas_ref[...] = jnp.dot(h2, as_ref[...], preferred_element_type=F32)
    oad_ref[...] = jnp.dot(h2, ad_ref[...], preferred_element_type=F32)


def _tc_final(op2, dp2, b2r, pmat):
    grid = (NROW // RB,)
    return pl.pallas_call(
        _final_body,
        grid=grid,
        in_specs=[
            pl.BlockSpec((NC, RB, NOUT), lambda i: (0, i, 0)),
            pl.BlockSpec((NC, RB, 16), lambda i: (0, i, 0)),
            pl.BlockSpec((1, NOUT), lambda i: (0, 0)),
            pl.BlockSpec((NOUT, NOUT), lambda i: (0, 0)),
        ],
        out_specs=pl.BlockSpec((RB, NOUT), lambda i: (i, 0)),
        out_shape=jax.ShapeDtypeStruct((NROW, NOUT), F32),
    )(op2, dp2, b2r, pmat)


def _final_body(op_ref, dp_ref, b2_ref, pm_ref, o_ref):
    r = 1.0 / (dp_ref[0, :, 0:1] + dp_ref[1, :, 0:1] + 1e-16)
    y = jnp.dot((op_ref[0] + op_ref[1]) * r, pm_ref[...],
                preferred_element_type=F32) + b2_ref[...]
    m = jnp.max(y, axis=1, keepdims=True)
    z = y - m
    o_ref[...] = z - jnp.log(jnp.sum(jnp.exp(z), axis=1, keepdims=True))


def kernel(x, edge_index, W1, att_src1, att_dst1, b1, W2, att_src2,
           att_dst2, b2):
    npad = NBLKT * EB - NEDGE
    pad = jnp.full((npad,), NROW - 1, jnp.int32)
    srcp = jnp.concatenate([edge_index[0].astype(jnp.int32), pad])
    dstp = jnp.concatenate([edge_index[1].astype(jnp.int32), pad])
    edg = jnp.stack([srcp.reshape(NBLKT, EB), dstp.reshape(NBLKT, EB)],
                    axis=1)

    eye816 = jnp.eye(NHEAD, 16, dtype=F32)
    A1s = (att_src1[:, :, None] * eye816[:, None, :]).reshape(NHEAD * NHID, 16)
    A1d = (att_dst1[:, :, None] * eye816[:, None, :]).reshape(NHEAD * NHID, 16)
    A2s = att_src2.T * jnp.eye(1, 16, dtype=F32)
    A2d = att_dst2.T * jnp.eye(1, 16, dtype=F32)
    permc = jnp.array([2 * k for k in range(16)]
                      + [2 * k + 1 for k in range(16)]
                      + [32 + 2 * k for k in range(16)]
                      + [33 + 2 * k for k in range(16)], dtype=jnp.int32)
    b1r = b1.reshape(8, 64)[:, permc]
    W2r = W2.reshape(8, 64, NOUT)[:, permc, :]
    b2r = b2.reshape(1, NOUT)
    xp = jnp.pad(x, ((0, NROW - NNODE), (0, 0)))

    h4, as1, ad1 = _tc_dense1(xp, W1, A1s, A1d)
    w1, dp1 = _sc_logits(as1, ad1, edg)
    hs1 = tuple(
        jax.lax.bitcast_convert_type(h4[c].reshape(NROW, 32, 2), jnp.int32)
        for c in range(8))
    lanes1 = lambda c: [c] * 4
    op1 = _sc_agg(hs1, w1, edg, 64, lanes1, bf16=True)

    h2, as2, ad2 = _tc_dense2(op1, dp1, b1r, W2r, A2s, A2d)
    w2, dp2 = _sc_logits(as2, ad2, edg)
    lanes2 = lambda c: [0] * 4
    h2i = jax.lax.bitcast_convert_type(h2.reshape(NROW, 32, 2), jnp.int32)
    op2 = _sc_agg((h2i,), w2, edg, NOUT, lanes2, bf16=True)

    pmat = jnp.zeros((NOUT, NOUT), F32).at[jnp.arange(NOUT), permc].set(1.0)
    return _tc_final(op2.reshape(NC, NROW, NOUT), dp2, b2r, pmat)[:NNODE]
